# baseline (speedup 1.0000x reference)
import jax
import jax.numpy as jnp
from jax import lax
from jax.experimental import pallas as pl
from jax.experimental.pallas import tpu as pltpu
from jax.experimental.pallas import tpu_sc as plsc

N = 10000
NPAD = 10240
D = 128
E = 320000
CH = 128
ZR = NPAD // 16
RB = 1000
MB = 1024
NG = 10
HR = 5120
AR = 5184
AZ = AR // 8
WB = HR // 16

EPT = E // 16
NCH = 157
PH1 = 80

_mesh = plsc.VectorSubcoreMesh(core_axis_name="c", subcore_axis_name="s")


def _hist_kernel(eall, out, hist2, ibs, ibd):
    c = lax.axis_index("c")
    s = lax.axis_index("s")
    pltpu.sync_copy(eall.at[c, 0, s], ibs)
    pltpu.sync_copy(eall.at[c, 1, s], ibd)
    ones16 = jnp.ones((16,), jnp.float32)

    for a, ib in ((0, ibs), (1, ibd)):
        def zero_r(i, carry):
            def zero_k(k, carry2):
                hist2[i, pl.ds(k * 16, 16)] = jnp.zeros((16,), jnp.float32)
                return carry2

            return lax.fori_loop(0, CH // 16, zero_k, carry)

        lax.fori_loop(0, NPAD // CH, zero_r, 0)

        def scat_j(j, carry, _ib=ib):
            def scat_k(k, carry2):
                idx = _ib[j, pl.ds(k * 16, 16)]
                ri = lax.shift_right_logical(idx, 7)
                ci = lax.bitwise_and(idx, 127)
                plsc.addupdate_scatter(hist2, [ri, ci], ones16)
                return carry2

            return lax.fori_loop(0, CH // 16, scat_k, carry)

        lax.fori_loop(0, NCH, scat_j, 0)
        pltpu.sync_copy(hist2, out.at[c, a, s])


_hist_call = pl.kernel(
    _hist_kernel,
    out_type=jax.ShapeDtypeStruct((2, 2, 16, NPAD // CH, CH), jnp.float32),
    mesh=_mesh,
    scratch_types=[
        pltpu.VMEM((NPAD // CH, CH), jnp.float32),
        pltpu.VMEM((NCH, CH), jnp.int32),
        pltpu.VMEM((NCH, CH), jnp.int32),
    ],
    compiler_params=pltpu.CompilerParams(needs_layout_passes=False),
)


def _agg_kernel(h_hbm, edges, zeros_hbm, out, accum, ibuf, r0, r1, r2,
                sem_g, sem_s):
    c = lax.axis_index("c")
    s = lax.axis_index("s")
    base = c * HR

    @pl.when(s < 8)
    def _zero():
        pltpu.sync_copy(zeros_hbm.at[pl.ds(s * AZ, AZ)],
                        accum.at[pl.ds(s * AZ, AZ)])

    def gath(j, buf):
        pltpu.async_copy(h_hbm.at[ibuf.at[0, j]], buf, sem_g)

    def scat(j, buf):
        pltpu.async_copy(buf, accum.at[ibuf.at[1, j]], sem_s, add=True)

    def wait_g():
        pltpu.make_async_copy(h_hbm.at[pl.ds(0, CH)], r0, sem_g).wait()

    def wait_s():
        pltpu.make_async_copy(r0, accum.at[pl.ds(0, CH)], sem_s).wait()

    first = True
    for jb, n in ((0, PH1), (PH1, NCH - PH1)):
        pltpu.sync_copy(edges.at[0, s, pl.ds(jb, n)], ibuf.at[0, pl.ds(0, n)])
        pltpu.sync_copy(edges.at[1, s, pl.ds(jb, n)], ibuf.at[1, pl.ds(0, n)])

        def rew_j(j, carry):
            def rew_k(k, carry2):
                d = ibuf[1, j, pl.ds(k * 16, 16)]
                loc = d - base
                ok = (loc >= 0) & (loc < HR)
                ibuf[1, j, pl.ds(k * 16, 16)] = jnp.where(ok, loc, HR)
                return carry2

            return lax.fori_loop(0, CH // 16, rew_k, carry)

        lax.fori_loop(0, n, rew_j, 0)
        if first:
            plsc.subcore_barrier()
            first = False

        gath(0, r0)
        gath(1, r1)

        def body(p, carry):
            j0 = 3 * p
            wait_g()
            scat(j0, r0)

            @pl.when(p > 0)
            def _ws():
                wait_s()

            gath(j0 + 2, r2)
            wait_g()
            scat(j0 + 1, r1)
            wait_s()
            gath(j0 + 3, r0)
            wait_g()
            scat(j0 + 2, r2)
            wait_s()
            gath(j0 + 4, r1)
            return carry

        lax.fori_loop(0, (n - 2) // 3, body, 0)
        wait_g()
        scat(n - 2, r0)
        wait_s()
        wait_g()
        scat(n - 1, r1)
        wait_s()
        wait_s()

    plsc.subcore_barrier()
    pltpu.sync_copy(accum.at[pl.ds(s * WB, WB)],
                    out.at[pl.ds(base + s * WB, WB)])


_agg_call = pl.kernel(
    _agg_kernel,
    out_type=jax.ShapeDtypeStruct((NPAD, D), jnp.float32),
    mesh=_mesh,
    scratch_types=[
        pltpu.VMEM_SHARED((AR, D), jnp.float32),
        pltpu.VMEM((2, PH1, CH), jnp.int32),
        pltpu.VMEM((CH, D), jnp.float32),
        pltpu.VMEM((CH, D), jnp.float32),
        pltpu.VMEM((CH, D), jnp.float32),
        pltpu.SemaphoreType.DMA,
        pltpu.SemaphoreType.DMA,
    ],
)


def _deg_scale(hp):
    deg = jnp.sum(hp, axis=0)[:, None]
    return lax.rsqrt(jnp.maximum(deg, 1.0))


def _scale_kernel(x_ref, hp_ref, o_ref):
    o_ref[...] = x_ref[...] * _deg_scale(hp_ref[...])


_scale_call = pl.pallas_call(
    _scale_kernel,
    out_shape=jax.ShapeDtypeStruct((N, D), jnp.float32),
    grid=(NG,),
    in_specs=[
        pl.BlockSpec((MB, D), lambda i: (i, 0)),
        pl.BlockSpec((16, MB), lambda i: (0, i)),
    ],
    out_specs=pl.BlockSpec((MB, D), lambda i: (i, 0)),
)


def _mm0_kernel(agg_ref, hd_ref, hs_ref, w_ref, b_ref, o_ref):
    z = agg_ref[...] * _deg_scale(hd_ref[...])
    y = jnp.dot(z, w_ref[...], preferred_element_type=jnp.float32)
    y = jnp.maximum(y + b_ref[...], 0.0)
    o_ref[...] = y * _deg_scale(hs_ref[...])


_mm0_call = pl.pallas_call(
    _mm0_kernel,
    out_shape=jax.ShapeDtypeStruct((N, D), jnp.float32),
    grid=(NG,),
    in_specs=[
        pl.BlockSpec((MB, D), lambda i: (i, 0)),
        pl.BlockSpec((16, MB), lambda i: (0, i)),
        pl.BlockSpec((16, MB), lambda i: (0, i)),
        pl.BlockSpec((D, D), lambda i: (0, 0)),
        pl.BlockSpec((1, D), lambda i: (0, 0)),
    ],
    out_specs=pl.BlockSpec((MB, D), lambda i: (i, 0)),
)


def _mm1_kernel(agg_ref, hd_ref, w_ref, b_ref, o_ref):
    z = agg_ref[...] * _deg_scale(hd_ref[...])
    y = jnp.dot(z, w_ref[...], preferred_element_type=jnp.float32)
    o_ref[...] = jnp.maximum(y + b_ref[...], 0.0)


_mm1_call = pl.pallas_call(
    _mm1_kernel,
    out_shape=jax.ShapeDtypeStruct((N, D), jnp.float32),
    grid=(NG,),
    in_specs=[
        pl.BlockSpec((MB, D), lambda i: (i, 0)),
        pl.BlockSpec((16, MB), lambda i: (0, i)),
        pl.BlockSpec((D, D), lambda i: (0, 0)),
        pl.BlockSpec((1, D), lambda i: (0, 0)),
    ],
    out_specs=pl.BlockSpec((MB, D), lambda i: (i, 0)),
)


def _prep_edges(ei, pad_src, pad_dst):
    padw = NCH * CH - EPT
    src = ei[0].reshape(16, EPT)
    dst = ei[1].reshape(16, EPT)
    srcp = jnp.pad(src, ((0, 0), (0, padw)),
                   constant_values=pad_src).reshape(16, NCH, CH)
    dstp = jnp.pad(dst, ((0, 0), (0, padw)),
                   constant_values=pad_dst).reshape(16, NCH, CH)
    return jnp.stack([srcp, dstp])


def kernel(x, edge_index0, edge_index1, num_dst0, num_dst1, W0, b0, W1, b1):
    ehist = jnp.stack([_prep_edges(edge_index0, N, N),
                       _prep_edges(edge_index1, N, N)])
    eagg0 = _prep_edges(edge_index0, 0, N)
    eagg1 = _prep_edges(edge_index1, 0, N)
    zeros = jnp.zeros((AR, D), jnp.float32)

    hists = _hist_call(ehist).reshape(2, 2, 16, NPAD)
    h = _scale_call(x, hists[0, 0])
    agg0 = _agg_call(h, eagg0, zeros)
    h1 = _mm0_call(agg0, hists[0, 1], hists[1, 0], W0, b0.reshape(1, D))
    agg1 = _agg_call(h1, eagg1, zeros)
    out = _mm1_call(agg1, hists[1, 1], W1, b1.reshape(1, D))
    return out


# speedup vs baseline: 1.2944x; 1.2944x over previous
import jax
import jax.numpy as jnp
from jax import lax
from jax.experimental import pallas as pl
from jax.experimental.pallas import tpu as pltpu
from jax.experimental.pallas import tpu_sc as plsc

N = 10000
NPAD = 10240
D = 128
E = 320000
CH = 128
ZR = NPAD // 16
RB = 1000
MB = 1024
NG = 10
NCHA = 79
PA = 40

EPT = E // 16
NCH = 157

_mesh = plsc.VectorSubcoreMesh(core_axis_name="c", subcore_axis_name="s")


def _hist_kernel(eall, out, hist2, ibs, ibd):
    c = lax.axis_index("c")
    s = lax.axis_index("s")
    pltpu.sync_copy(eall.at[c, 0, s], ibs)
    pltpu.sync_copy(eall.at[c, 1, s], ibd)
    ones16 = jnp.ones((16,), jnp.float32)

    for a, ib in ((0, ibs), (1, ibd)):
        def zero_r(i, carry):
            def zero_k(k, carry2):
                hist2[i, pl.ds(k * 16, 16)] = jnp.zeros((16,), jnp.float32)
                return carry2

            return lax.fori_loop(0, CH // 16, zero_k, carry)

        lax.fori_loop(0, NPAD // CH, zero_r, 0)

        def scat_j(j, carry, _ib=ib):
            def scat_k(k, carry2):
                idx = _ib[j, pl.ds(k * 16, 16)]
                ri = lax.shift_right_logical(idx, 7)
                ci = lax.bitwise_and(idx, 127)
                plsc.addupdate_scatter(hist2, [ri, ci], ones16)
                return carry2

            return lax.fori_loop(0, CH // 16, scat_k, carry)

        lax.fori_loop(0, NCH, scat_j, 0)
        pltpu.sync_copy(hist2, out.at[c, a, s])


_hist_call = pl.kernel(
    _hist_kernel,
    out_type=jax.ShapeDtypeStruct((2, 2, 16, NPAD // CH, CH), jnp.float32),
    mesh=_mesh,
    scratch_types=[
        pltpu.VMEM((NPAD // CH, CH), jnp.float32),
        pltpu.VMEM((NCH, CH), jnp.int32),
        pltpu.VMEM((NCH, CH), jnp.int32),
    ],
    compiler_params=pltpu.CompilerParams(needs_layout_passes=False),
)


def _agg_kernel(h_hbm, edges, zeros_hbm, out, accum, ibuf, r0, r1,
                sem_g, sem_s):
    c = lax.axis_index("c")
    s = lax.axis_index("s")
    wid = c * 16 + s
    pltpu.sync_copy(zeros_hbm.at[pl.ds(s * ZR, ZR)],
                    accum.at[pl.ds(s * ZR, ZR)])

    def gath(j, buf):
        pltpu.async_copy(h_hbm.at[ibuf.at[0, j]], buf, sem_g)

    def scat(j, buf):
        pltpu.async_copy(buf, accum.at[ibuf.at[1, j]], sem_s, add=True)

    def wait_g():
        pltpu.make_async_copy(h_hbm.at[pl.ds(0, CH)], r0, sem_g).wait()

    def wait_s():
        pltpu.make_async_copy(r0, accum.at[pl.ds(0, CH)], sem_s).wait()

    first = True
    for jb, n in ((0, PA), (PA, NCHA - PA)):
        pltpu.sync_copy(edges.at[0, wid, pl.ds(jb, n)],
                        ibuf.at[0, pl.ds(0, n)])
        pltpu.sync_copy(edges.at[1, wid, pl.ds(jb, n)],
                        ibuf.at[1, pl.ds(0, n)])
        if first:
            plsc.subcore_barrier()
            first = False

        gath(0, r0)

        def body(p, carry, _n=n):
            j0 = 2 * p
            wait_g()

            @pl.when(p > 0)
            def _ws():
                wait_s()

            scat(j0, r0)
            gath(j0 + 1, r1)
            wait_g()
            wait_s()
            scat(j0 + 1, r1)

            @pl.when(j0 + 2 < _n)
            def _g2():
                gath(j0 + 2, r0)

            return carry

        lax.fori_loop(0, n // 2, body, 0)
        if n % 2:
            wait_g()
            wait_s()
            scat(n - 1, r0)
            wait_s()
        else:
            wait_s()

    plsc.subcore_barrier()

    @pl.when(s < 10)
    def _writeback():
        pltpu.sync_copy(accum.at[pl.ds(s * RB, RB)],
                        out.at[c, pl.ds(s * RB, RB)])


_agg_call = pl.kernel(
    _agg_kernel,
    out_type=jax.ShapeDtypeStruct((2, N, D), jnp.float32),
    mesh=_mesh,
    scratch_types=[
        pltpu.VMEM_SHARED((NPAD, D), jnp.float32),
        pltpu.VMEM((2, PA, CH), jnp.int32),
        pltpu.VMEM((CH, D), jnp.float32),
        pltpu.VMEM((CH, D), jnp.float32),
        pltpu.SemaphoreType.DMA,
        pltpu.SemaphoreType.DMA,
    ],
)


def _deg_scale(hp):
    deg = jnp.sum(hp, axis=0)[:, None]
    return lax.rsqrt(jnp.maximum(deg, 1.0))


def _scale_kernel(x_ref, hp_ref, o_ref):
    o_ref[...] = x_ref[...] * _deg_scale(hp_ref[...])


_scale_call = pl.pallas_call(
    _scale_kernel,
    out_shape=jax.ShapeDtypeStruct((N, D), jnp.float32),
    grid=(NG,),
    in_specs=[
        pl.BlockSpec((MB, D), lambda i: (i, 0)),
        pl.BlockSpec((16, MB), lambda i: (0, i)),
    ],
    out_specs=pl.BlockSpec((MB, D), lambda i: (i, 0)),
)


def _mm0_kernel(agg_ref, hd_ref, hs_ref, w_ref, b_ref, o_ref):
    z = (agg_ref[0] + agg_ref[1]) * _deg_scale(hd_ref[...])
    y = jnp.dot(z, w_ref[...], preferred_element_type=jnp.float32)
    y = jnp.maximum(y + b_ref[...], 0.0)
    o_ref[...] = y * _deg_scale(hs_ref[...])


_mm0_call = pl.pallas_call(
    _mm0_kernel,
    out_shape=jax.ShapeDtypeStruct((N, D), jnp.float32),
    grid=(NG,),
    in_specs=[
        pl.BlockSpec((2, MB, D), lambda i: (0, i, 0)),
        pl.BlockSpec((16, MB), lambda i: (0, i)),
        pl.BlockSpec((16, MB), lambda i: (0, i)),
        pl.BlockSpec((D, D), lambda i: (0, 0)),
        pl.BlockSpec((1, D), lambda i: (0, 0)),
    ],
    out_specs=pl.BlockSpec((MB, D), lambda i: (i, 0)),
)


def _mm1_kernel(agg_ref, hd_ref, w_ref, b_ref, o_ref):
    z = (agg_ref[0] + agg_ref[1]) * _deg_scale(hd_ref[...])
    y = jnp.dot(z, w_ref[...], preferred_element_type=jnp.float32)
    o_ref[...] = jnp.maximum(y + b_ref[...], 0.0)


_mm1_call = pl.pallas_call(
    _mm1_kernel,
    out_shape=jax.ShapeDtypeStruct((N, D), jnp.float32),
    grid=(NG,),
    in_specs=[
        pl.BlockSpec((2, MB, D), lambda i: (0, i, 0)),
        pl.BlockSpec((16, MB), lambda i: (0, i)),
        pl.BlockSpec((D, D), lambda i: (0, 0)),
        pl.BlockSpec((1, D), lambda i: (0, 0)),
    ],
    out_specs=pl.BlockSpec((MB, D), lambda i: (i, 0)),
)


def _prep_edges(ei, nsplit, nch, pad_src, pad_dst):
    ept = E // nsplit
    padw = nch * CH - ept
    src = ei[0].reshape(nsplit, ept)
    dst = ei[1].reshape(nsplit, ept)
    srcp = jnp.pad(src, ((0, 0), (0, padw)),
                   constant_values=pad_src).reshape(nsplit, nch, CH)
    dstp = jnp.pad(dst, ((0, 0), (0, padw)),
                   constant_values=pad_dst).reshape(nsplit, nch, CH)
    return jnp.stack([srcp, dstp])


def kernel(x, edge_index0, edge_index1, num_dst0, num_dst1, W0, b0, W1, b1):
    ehist = jnp.stack([_prep_edges(edge_index0, 16, NCH, N, N),
                       _prep_edges(edge_index1, 16, NCH, N, N)])
    eagg0 = _prep_edges(edge_index0, 32, NCHA, 0, N)
    eagg1 = _prep_edges(edge_index1, 32, NCHA, 0, N)
    zeros = jnp.zeros((NPAD, D), jnp.float32)

    hists = _hist_call(ehist).reshape(2, 2, 16, NPAD)
    h = _scale_call(x, hists[0, 0])
    agg0 = _agg_call(h, eagg0, zeros)
    h1 = _mm0_call(agg0, hists[0, 1], hists[1, 0], W0, b0.reshape(1, D))
    agg1 = _agg_call(h1, eagg1, zeros)
    out = _mm1_call(agg1, hists[1, 1], W1, b1.reshape(1, D))
    return out
